# baseline (speedup 1.0000x reference)
import functools

import jax
import jax.numpy as jnp
from jax import lax
from jax.experimental import pallas as pl
from jax.experimental.pallas import tpu as pltpu
from jax.experimental.pallas import tpu_sc as plsc

_ROWS = 16384
_COLS = 200
_N = _ROWS * _COLS
_NW = 32
_PER_W = _N // _NW
_CHUNK = 25600
_NCHUNKS = _PER_W // _CHUNK


def _lookup_body(x_hbm, w_hbm, out_hbm, w_v, idx_v, rows_v):
    wid = lax.axis_index("s") * 2 + lax.axis_index("c")
    base = wid * _PER_W
    pltpu.sync_copy(w_hbm, w_v)
    w16 = w_v[...]

    for c in range(_NCHUNKS):
        off = base + c * _CHUNK
        pltpu.sync_copy(x_hbm.at[pl.ds(off, _CHUNK)], idx_v)

        def inner(j, _):
            iv = idx_v[pl.ds(j * 16, 16)]
            rows_v[pl.ds(j * 16, 16)] = lax.gather(
                w16, iv[:, None],
                lax.GatherDimensionNumbers(
                    offset_dims=(), collapsed_slice_dims=(0,),
                    start_index_map=(0,)),
                slice_sizes=(1,),
                mode=lax.GatherScatterMode.PROMISE_IN_BOUNDS)
            return 0

        lax.fori_loop(0, _CHUNK // 16, inner, 0)
        pltpu.sync_copy(rows_v, out_hbm.at[pl.ds(off, _CHUNK)])


@jax.jit
def kernel(x, weight):
    x_flat = x.reshape(-1).astype(jnp.int32)
    w_flat = jnp.pad(weight.reshape(-1).astype(jnp.float32), (0, 12))
    mesh = plsc.VectorSubcoreMesh(core_axis_name="c", subcore_axis_name="s")
    out = pl.kernel(
        _lookup_body,
        mesh=mesh,
        out_type=jax.ShapeDtypeStruct((_N,), jnp.float32),
        scratch_types=[
            pltpu.VMEM((16,), jnp.float32),
            pltpu.VMEM((_CHUNK,), jnp.int32),
            pltpu.VMEM((_CHUNK,), jnp.float32),
        ],
    )(x_flat, w_flat)
    return out.reshape(_ROWS, _COLS, 1)


# speedup vs baseline: 1.1882x; 1.1882x over previous
import jax
import jax.numpy as jnp
from jax import lax
from jax.experimental import pallas as pl
from jax.experimental.pallas import tpu as pltpu
from jax.experimental.pallas import tpu_sc as plsc

_ROWS = 16384
_COLS = 200
_N = _ROWS * _COLS
_NW = 32
_PER_W = _N // _NW
_CHUNK = 12800
_NCHUNKS = _PER_W // _CHUNK
_UNROLL = 8


def _gather16(w16, iv):
    return lax.gather(
        w16, iv[:, None],
        lax.GatherDimensionNumbers(
            offset_dims=(), collapsed_slice_dims=(0,), start_index_map=(0,)),
        slice_sizes=(1,),
        mode=lax.GatherScatterMode.PROMISE_IN_BOUNDS)


def _lookup_body(x_hbm, w_hbm, out_hbm, w_v,
                 idx0, idx1, rows0, rows1, isem0, isem1, osem0, osem1):
    wid = lax.axis_index("s") * 2 + lax.axis_index("c")
    base = wid * _PER_W
    pltpu.sync_copy(w_hbm, w_v)
    w16 = w_v[...]

    idx = (idx0, idx1)
    rows = (rows0, rows1)
    isem = (isem0, isem1)
    osem = (osem0, osem1)
    in_h = [None, None]
    out_h = [None, None]

    def off(c):
        return base + c * _CHUNK

    in_h[0] = pltpu.async_copy(x_hbm.at[pl.ds(off(0), _CHUNK)], idx[0], isem[0])
    for c in range(_NCHUNKS):
        b = c & 1
        in_h[b].wait()
        if c + 1 < _NCHUNKS:
            in_h[1 - b] = pltpu.async_copy(
                x_hbm.at[pl.ds(off(c + 1), _CHUNK)], idx[1 - b], isem[1 - b])
        if out_h[b] is not None:
            out_h[b].wait()

        def inner(j, _, b=b):
            jb = j * (16 * _UNROLL)
            for u in range(_UNROLL):
                s = jb + u * 16
                rows[b][pl.ds(s, 16)] = _gather16(w16, idx[b][pl.ds(s, 16)])
            return 0

        lax.fori_loop(0, _CHUNK // (16 * _UNROLL), inner, 0)
        out_h[b] = pltpu.async_copy(
            rows[b], out_hbm.at[pl.ds(off(c), _CHUNK)], osem[b])
    out_h[0].wait()
    out_h[1].wait()


@jax.jit
def kernel(x, weight):
    x_flat = x.reshape(-1).astype(jnp.int32)
    w_flat = jnp.pad(weight.reshape(-1).astype(jnp.float32), (0, 12))
    mesh = plsc.VectorSubcoreMesh(core_axis_name="c", subcore_axis_name="s")
    out = pl.kernel(
        _lookup_body,
        mesh=mesh,
        out_type=jax.ShapeDtypeStruct((_N,), jnp.float32),
        scratch_types=[
            pltpu.VMEM((16,), jnp.float32),
            pltpu.VMEM((_CHUNK,), jnp.int32),
            pltpu.VMEM((_CHUNK,), jnp.int32),
            pltpu.VMEM((_CHUNK,), jnp.float32),
            pltpu.VMEM((_CHUNK,), jnp.float32),
            pltpu.SemaphoreType.DMA,
            pltpu.SemaphoreType.DMA,
            pltpu.SemaphoreType.DMA,
            pltpu.SemaphoreType.DMA,
        ],
    )(x_flat, w_flat)
    return out.reshape(_ROWS, _COLS, 1)


# speedup vs baseline: 1.9144x; 1.6112x over previous
import jax
import jax.numpy as jnp
from jax.experimental import pallas as pl
from jax.experimental.pallas import tpu as pltpu

_ROWS = 16384
_COLS = 200
_BLK = 512
_GRID = _ROWS // _BLK


def _tc_body(w_ref, x_ref, out_ref):
    xb = x_ref[...]
    w0 = w_ref[0, 0]
    w1 = w_ref[0, 1]
    w2 = w_ref[0, 2]
    w3 = w_ref[0, 3]
    lo = jnp.where(xb == 1, w1, w0)
    hi = jnp.where(xb == 3, w3, w2)
    out_ref[...] = jnp.where(xb >= 2, hi, lo)


@jax.jit
def kernel(x, weight):
    w_row = weight.reshape(1, 4).astype(jnp.float32)
    out = pl.pallas_call(
        _tc_body,
        grid=(_GRID,),
        in_specs=[
            pl.BlockSpec(memory_space=pltpu.SMEM),
            pl.BlockSpec((_BLK, _COLS), lambda i: (i, 0)),
        ],
        out_specs=pl.BlockSpec((_BLK, _COLS), lambda i: (i, 0)),
        out_shape=jax.ShapeDtypeStruct((_ROWS, _COLS), jnp.float32),
    )(w_row, x)
    return out.reshape(_ROWS, _COLS, 1)


# speedup vs baseline: 4.9481x; 2.5847x over previous
import jax
import jax.numpy as jnp
from jax.experimental import pallas as pl
from jax.experimental.pallas import tpu as pltpu

_ROWS = 16384
_COLS = 200
_RB = 8
_GRID = _COLS // _RB


def _tc_body(w_ref, x_ref, out_ref, acc_ref, sem):
    i = pl.program_id(0)
    xb = x_ref[...]
    w0 = w_ref[0, 0]
    w1 = w_ref[0, 1]
    w2 = w_ref[0, 2]
    w3 = w_ref[0, 3]
    lo = jnp.where(xb == 1, w1, w0)
    hi = jnp.where(xb == 3, w3, w2)
    acc_ref[...] = jnp.where(xb >= 2, hi, lo)
    cps = []
    for r in range(_RB):
        cps.append(pltpu.make_async_copy(
            acc_ref.at[r], out_ref.at[i * _RB + r, 0], sem))
        cps[-1].start()
    for cp in cps:
        cp.wait()


@jax.jit
def kernel(x, weight):
    w_row = weight.reshape(1, 4).astype(jnp.float32)
    xt = x.T
    out_lin = pl.pallas_call(
        _tc_body,
        grid=(_GRID,),
        in_specs=[
            pl.BlockSpec(memory_space=pltpu.SMEM),
            pl.BlockSpec((_RB, _ROWS), lambda i: (i, 0)),
        ],
        out_specs=pl.BlockSpec(memory_space=pl.ANY),
        out_shape=jax.ShapeDtypeStruct((_COLS, 1, _ROWS), jnp.float32),
        scratch_shapes=[
            pltpu.VMEM((_RB, _ROWS), jnp.float32),
            pltpu.SemaphoreType.DMA,
        ],
    )(w_row, xt)
    return jnp.transpose(out_lin, (2, 0, 1))


# speedup vs baseline: 5.5417x; 1.1200x over previous
import jax
import jax.numpy as jnp
from jax import lax
from jax.experimental import pallas as pl
from jax.experimental.pallas import tpu as pltpu

_ROWS = 16384
_COLS = 200
_RB = 8
_GRID = _COLS // _RB


def _tc_body(w_ref, x_ref, out_ref, acc_ref, sem):
    i = pl.program_id(0)
    b = lax.rem(i, 2)

    def dma(step, buf, r):
        return pltpu.make_async_copy(
            acc_ref.at[buf, r], out_ref.at[step * _RB + r, 0], sem)

    @pl.when(i > 0)
    def _():
        for r in range(_RB):
            dma(i - 1, 1 - b, r).wait()

    xb = x_ref[...]
    w0 = w_ref[0, 0]
    w1 = w_ref[0, 1]
    w2 = w_ref[0, 2]
    w3 = w_ref[0, 3]
    lo = jnp.where(xb == 1, w1, w0)
    hi = jnp.where(xb == 3, w3, w2)
    acc_ref[b] = jnp.where(xb >= 2, hi, lo)
    for r in range(_RB):
        dma(i, b, r).start()

    @pl.when(i == _GRID - 1)
    def _():
        for r in range(_RB):
            dma(i, b, r).wait()


@jax.jit
def kernel(x, weight):
    w_row = weight.reshape(1, 4).astype(jnp.float32)
    xt = x.T
    out_lin = pl.pallas_call(
        _tc_body,
        grid=(_GRID,),
        in_specs=[
            pl.BlockSpec(memory_space=pltpu.SMEM),
            pl.BlockSpec((_RB, _ROWS), lambda i: (i, 0)),
        ],
        out_specs=pl.BlockSpec(memory_space=pl.ANY),
        out_shape=jax.ShapeDtypeStruct((_COLS, 1, _ROWS), jnp.float32),
        scratch_shapes=[
            pltpu.VMEM((2, _RB, _ROWS), jnp.float32),
            pltpu.SemaphoreType.DMA,
        ],
    )(w_row, xt)
    return jnp.transpose(out_lin, (2, 0, 1))
